# Initial kernel scaffold; baseline (speedup 1.0000x reference)
import jax
import jax.numpy as jnp
from jax.experimental import pallas as pl


def kernel(heatmaps_input, offsets_input):
    raise NotImplementedError("write your pallas kernel here")


# speedup vs baseline: 1.2638x; 1.2638x over previous
import jax
import jax.numpy as jnp
from jax import lax
from jax.experimental import pallas as pl

H = 512
W = 512
C = 17
BH = 16


def _decode_kernel(hm_ref, off_ref, out_ref):
    i = pl.program_id(0)
    hmv = hm_ref[...]
    offv = off_ref[...]

    m = jnp.max(hmv, axis=-1, keepdims=True)
    iota_c = lax.broadcasted_iota(jnp.int32, (BH, W, C), 2)
    a = jnp.min(jnp.where(hmv == m, iota_c, C), axis=-1, keepdims=True)
    score = jax.nn.sigmoid(m)

    iota34 = lax.broadcasted_iota(jnp.int32, (BH, W, 2 * C), 2)
    y_off = jnp.sum(jnp.where(iota34 == a, offv, 0.0), axis=-1, keepdims=True)
    x_off = jnp.sum(jnp.where(iota34 == a + C, offv, 0.0), axis=-1, keepdims=True)

    row = (i * BH + lax.broadcasted_iota(jnp.int32, (BH, W, 1), 0)).astype(jnp.float32)
    col = lax.broadcasted_iota(jnp.int32, (BH, W, 1), 1).astype(jnp.float32)
    xv = (col * 4.0 + x_off).astype(jnp.int32).astype(jnp.float32)
    yv = (row * 4.0 + y_off).astype(jnp.int32).astype(jnp.float32)

    out_ref[...] = jnp.concatenate([a.astype(jnp.float32), score, xv, yv], axis=-1)


def kernel(heatmaps_input, offsets_input):
    hm = heatmaps_input.reshape(H, W, C)
    off = offsets_input.reshape(H, W, 2 * C)
    out = pl.pallas_call(
        _decode_kernel,
        grid=(H // BH,),
        in_specs=[
            pl.BlockSpec((BH, W, C), lambda i: (i, 0, 0)),
            pl.BlockSpec((BH, W, 2 * C), lambda i: (i, 0, 0)),
        ],
        out_specs=pl.BlockSpec((BH, W, 4), lambda i: (i, 0, 0)),
        out_shape=jax.ShapeDtypeStruct((H, W, 4), jnp.float32),
    )(hm, off)
    return out.reshape(1, H * W, 4)
